# baseline (speedup 1.0000x reference)
import jax
import jax.numpy as jnp
from jax.experimental import pallas as pl
from jax.experimental.pallas import tpu as pltpu

_BATCH = 128
_FEATURES = 2048
_CLASSES = 16522
_BETA = 0.05
_KNN = 6

_CHUNK_A = 1536
_NUM_A = 9
_CHUNK_B = 512
_NUM_B = 5
_B_BASE = _NUM_A * _CHUNK_A
_TAIL_BASE = _B_BASE + _NUM_B * _CHUNK_B
_TAIL_BLOCK = 256
_NBUF_A = 3
_NBUF_B = 3
_NSPLIT = 4
_LOOKAHEAD = 2
_NSTEP = _NUM_A + _NUM_B
_NEG = -1e30


def _topk_update(topk_s, scores, tmax):
    block_top = [tmax]
    mj = tmax
    for _ in range(_KNN - 1):
        mj = jnp.max(jnp.where(scores < mj, scores, _NEG),
                     axis=1, keepdims=True)
        block_top.append(mj)
    merged = jnp.concatenate([topk_s[...]] + block_top, axis=1)
    new_top = []
    for _ in range(_KNN):
        mj = jnp.max(merged, axis=1, keepdims=True)
        new_top.append(mj)
        merged = jnp.where(merged == mj, _NEG, merged)
    pad = jnp.full((_BATCH, 8 - _KNN), _NEG, jnp.float32)
    topk_s[...] = jnp.concatenate(new_top + [pad], axis=1)


def _accumulate(scores, col, lab_ref, m_s, s_s, vlab_s, topk_s):
    tmax = jnp.max(scores, axis=1, keepdims=True)
    m_prev = m_s[...]
    m_new = jnp.maximum(m_prev, tmax)
    s_s[...] = s_s[...] * jnp.exp(m_prev - m_new) + jnp.sum(
        jnp.exp(scores - m_new), axis=1, keepdims=True)
    m_s[...] = m_new
    vlab_s[...] += jnp.sum(
        jnp.where(col == lab_ref[...], scores, 0.0), axis=1, keepdims=True)
    _topk_update(topk_s, scores, tmax)


def _body(x_ref, lab_ref, tail_ref, em_hbm, out_ref,
          xs_s, bufa_s, bufb_s, m_s, s_s, vlab_s, topk_s, sema, semb):
    i = pl.program_id(0)

    _H = _CHUNK_A // _NSPLIT

    def a_copy_ops(c, slot):
        return [
            pltpu.make_async_copy(
                em_hbm.at[pl.ds(c * _CHUNK_A + h * _H, _H)],
                bufa_s.at[slot, pl.ds(h * _H, _H)],
                sema.at[slot, h],
            )
            for h in range(_NSPLIT)
        ]

    def b_copy_op(b, slot):
        return pltpu.make_async_copy(
            em_hbm.at[pl.ds(_B_BASE + b * _CHUNK_B, _CHUNK_B)],
            bufb_s.at[slot],
            semb.at[slot],
        )

    def start_chunk(j):
        @pl.when(j < _NUM_A)
        def _():
            for op in a_copy_ops(j, jax.lax.rem(j, _NBUF_A)):
                op.start()

        @pl.when(j >= _NUM_A)
        def _():
            b = j - _NUM_A
            b_copy_op(b, jax.lax.rem(b, _NBUF_B)).start()

    @pl.when(i == 0)
    def _init():
        for c in range(_LOOKAHEAD):
            start_chunk(jnp.int32(c))
        xs_s[...] = x_ref[...] * (1.0 / _BETA)
        m_s[...] = jnp.full((_BATCH, 1), _NEG, jnp.float32)
        s_s[...] = jnp.zeros((_BATCH, 1), jnp.float32)
        vlab_s[...] = jnp.zeros((_BATCH, 1), jnp.float32)
        topk_s[...] = jnp.full((_BATCH, 8), _NEG, jnp.float32)
        t_scores = jax.lax.dot_general(
            xs_s[...], tail_ref[...],
            (((1,), (1,)), ((), ())),
            preferred_element_type=jnp.float32,
        )
        t_col = _TAIL_BASE + jax.lax.broadcasted_iota(
            jnp.int32, (_BATCH, _TAIL_BLOCK), 1)
        t_scores = jnp.where(t_col < _CLASSES, t_scores, _NEG)
        _accumulate(t_scores, t_col, lab_ref, m_s, s_s, vlab_s, topk_s)

    @pl.when(i + _LOOKAHEAD < _NSTEP)
    def _prefetch():
        start_chunk(i + _LOOKAHEAD)

    @pl.when(i < _NUM_A)
    def _compute_a():
        slot = jax.lax.rem(i, _NBUF_A)
        for op in a_copy_ops(i, slot):
            op.wait()
        scores = jax.lax.dot_general(
            xs_s[...], bufa_s[slot],
            (((1,), (1,)), ((), ())),
            preferred_element_type=jnp.float32,
        )
        col = i * _CHUNK_A + jax.lax.broadcasted_iota(
            jnp.int32, (_BATCH, _CHUNK_A), 1)
        _accumulate(scores, col, lab_ref, m_s, s_s, vlab_s, topk_s)

    @pl.when(i >= _NUM_A)
    def _compute_b():
        b = i - _NUM_A
        slot = jax.lax.rem(b, _NBUF_B)
        b_copy_op(b, slot).wait()
        scores = jax.lax.dot_general(
            xs_s[...], bufb_s[slot],
            (((1,), (1,)), ((), ())),
            preferred_element_type=jnp.float32,
        )
        col = _B_BASE + b * _CHUNK_B + jax.lax.broadcasted_iota(
            jnp.int32, (_BATCH, _CHUNK_B), 1)
        _accumulate(scores, col, lab_ref, m_s, s_s, vlab_s, topk_s)

    @pl.when(i == _NSTEP - 1)
    def _finish():
        lse = m_s[...] + jnp.log(s_s[...])
        top = topk_s[...]
        top_sum = jnp.sum(top[:, 0:_KNN], axis=1, keepdims=True)
        vlab = vlab_s[...]
        kth = top[:, _KNN - 1:_KNN]
        in_top = vlab >= kth
        s_dot = 2.0 * top_sum + jnp.where(in_top, vlab, 3.0 * vlab)
        m_tot = jnp.where(in_top, 13.0, 15.0)
        loss_rows = lse * m_tot - s_dot
        out_ref[0, 0] = jnp.sum(loss_rows) / _BATCH


@jax.jit
def _run(inputs, label, em):
    lab2d = label.reshape(_BATCH, 1).astype(jnp.int32)
    out = pl.pallas_call(
        _body,
        grid=(_NSTEP,),
        in_specs=[
            pl.BlockSpec((_BATCH, _FEATURES), lambda i: (0, 0)),
            pl.BlockSpec((_BATCH, 1), lambda i: (0, 0)),
            pl.BlockSpec((_TAIL_BLOCK, _FEATURES),
                         lambda i: (_TAIL_BASE // _TAIL_BLOCK, 0)),
            pl.BlockSpec(memory_space=pl.ANY),
        ],
        out_specs=pl.BlockSpec(memory_space=pltpu.SMEM),
        out_shape=jax.ShapeDtypeStruct((1, 1), jnp.float32),
        compiler_params=pltpu.CompilerParams(
            vmem_limit_bytes=120 * 1024 * 1024,
        ),
        scratch_shapes=[
            pltpu.VMEM((_BATCH, _FEATURES), jnp.float32),
            pltpu.VMEM((_NBUF_A, _CHUNK_A, _FEATURES), jnp.float32),
            pltpu.VMEM((_NBUF_B, _CHUNK_B, _FEATURES), jnp.float32),
            pltpu.VMEM((_BATCH, 1), jnp.float32),
            pltpu.VMEM((_BATCH, 1), jnp.float32),
            pltpu.VMEM((_BATCH, 1), jnp.float32),
            pltpu.VMEM((_BATCH, 8), jnp.float32),
            pltpu.SemaphoreType.DMA((_NBUF_A, _NSPLIT)),
            pltpu.SemaphoreType.DMA((_NBUF_B,)),
        ],
    )(inputs, lab2d, em, em)
    return out[0, 0]


def kernel(inputs, label, epoch, em):
    del epoch
    return _run(inputs, label, em)


# speedup vs baseline: 1.0746x; 1.0746x over previous
import jax
import jax.numpy as jnp
from jax.experimental import pallas as pl
from jax.experimental.pallas import tpu as pltpu

_BATCH = 128
_FEATURES = 2048
_CLASSES = 16522
_BETA = 0.05
_KNN = 6

_CHUNK = 2048
_NCHUNK = 8
_TAIL_BASE = _NCHUNK * _CHUNK
_TAIL_BLOCK = 256
_NBUF = 3
_NSPLIT = 4
_NEG = -1e30


def _topk_update(topk_s, scores, tmax):
    block_top = [tmax]
    mj = tmax
    for _ in range(_KNN - 1):
        mj = jnp.max(jnp.where(scores < mj, scores, _NEG),
                     axis=1, keepdims=True)
        block_top.append(mj)
    merged = jnp.concatenate([topk_s[...]] + block_top, axis=1)
    new_top = []
    for _ in range(_KNN):
        mj = jnp.max(merged, axis=1, keepdims=True)
        new_top.append(mj)
        merged = jnp.where(merged == mj, _NEG, merged)
    pad = jnp.full((_BATCH, 8 - _KNN), _NEG, jnp.float32)
    topk_s[...] = jnp.concatenate(new_top + [pad], axis=1)


def _accumulate(scores, col, lab_ref, m_s, s_s, vlab_s, topk_s):
    tmax = jnp.max(scores, axis=1, keepdims=True)
    m_prev = m_s[...]
    m_new = jnp.maximum(m_prev, tmax)
    s_s[...] = s_s[...] * jnp.exp(m_prev - m_new) + jnp.sum(
        jnp.exp(scores - m_new), axis=1, keepdims=True)
    m_s[...] = m_new
    vlab_s[...] += jnp.sum(
        jnp.where(col == lab_ref[...], scores, 0.0), axis=1, keepdims=True)
    _topk_update(topk_s, scores, tmax)


def _body(x_ref, lab_ref, tail_ref, em_hbm, out_ref,
          xs_s, buf_s, m_s, s_s, vlab_s, topk_s, sem):
    i = pl.program_id(0)

    _H = _CHUNK // _NSPLIT

    def copy_ops(c, slot):
        return [
            pltpu.make_async_copy(
                em_hbm.at[pl.ds(c * _CHUNK + h * _H, _H)],
                buf_s.at[slot, pl.ds(h * _H, _H)],
                sem.at[slot, h],
            )
            for h in range(_NSPLIT)
        ]

    def start_copy(c, slot):
        for op in copy_ops(c, slot):
            op.start()

    def wait_copy(c, slot):
        for op in copy_ops(c, slot):
            op.wait()

    @pl.when(i == 0)
    def _init():
        for c in range(_NBUF - 1):
            start_copy(jnp.int32(c), jnp.int32(c))
        xs_s[...] = x_ref[...] * (1.0 / _BETA)
        m_s[...] = jnp.full((_BATCH, 1), _NEG, jnp.float32)
        s_s[...] = jnp.zeros((_BATCH, 1), jnp.float32)
        vlab_s[...] = jnp.zeros((_BATCH, 1), jnp.float32)
        topk_s[...] = jnp.full((_BATCH, 8), _NEG, jnp.float32)
        t_scores = jax.lax.dot_general(
            xs_s[...], tail_ref[...],
            (((1,), (1,)), ((), ())),
            preferred_element_type=jnp.float32,
        )
        t_col = _TAIL_BASE + jax.lax.broadcasted_iota(
            jnp.int32, (_BATCH, _TAIL_BLOCK), 1)
        t_scores = jnp.where(t_col < _CLASSES, t_scores, _NEG)
        _accumulate(t_scores, t_col, lab_ref, m_s, s_s, vlab_s, topk_s)

    @pl.when(i + _NBUF - 1 < _NCHUNK)
    def _prefetch():
        c = i + _NBUF - 1
        start_copy(c, jax.lax.rem(c, _NBUF))

    slot = jax.lax.rem(i, _NBUF)
    wait_copy(i, slot)

    scores = jax.lax.dot_general(
        xs_s[...], buf_s[slot],
        (((1,), (1,)), ((), ())),
        preferred_element_type=jnp.float32,
    )
    col = i * _CHUNK + jax.lax.broadcasted_iota(
        jnp.int32, (_BATCH, _CHUNK), 1)
    _accumulate(scores, col, lab_ref, m_s, s_s, vlab_s, topk_s)

    @pl.when(i == _NCHUNK - 1)
    def _finish():
        lse = m_s[...] + jnp.log(s_s[...])
        top = topk_s[...]
        top_sum = jnp.sum(top[:, 0:_KNN], axis=1, keepdims=True)
        vlab = vlab_s[...]
        kth = top[:, _KNN - 1:_KNN]
        in_top = vlab >= kth
        s_dot = 2.0 * top_sum + jnp.where(in_top, vlab, 3.0 * vlab)
        m_tot = jnp.where(in_top, 13.0, 15.0)
        loss_rows = lse * m_tot - s_dot
        out_ref[0, 0] = jnp.sum(loss_rows) / _BATCH


@jax.jit
def _run(inputs, label, em):
    lab2d = label.reshape(_BATCH, 1).astype(jnp.int32)
    out = pl.pallas_call(
        _body,
        grid=(_NCHUNK,),
        in_specs=[
            pl.BlockSpec((_BATCH, _FEATURES), lambda i: (0, 0)),
            pl.BlockSpec((_BATCH, 1), lambda i: (0, 0)),
            pl.BlockSpec((_TAIL_BLOCK, _FEATURES),
                         lambda i: (_TAIL_BASE // _TAIL_BLOCK, 0)),
            pl.BlockSpec(memory_space=pl.ANY),
        ],
        out_specs=pl.BlockSpec(memory_space=pltpu.SMEM),
        out_shape=jax.ShapeDtypeStruct((1, 1), jnp.float32),
        compiler_params=pltpu.CompilerParams(
            vmem_limit_bytes=120 * 1024 * 1024,
        ),
        scratch_shapes=[
            pltpu.VMEM((_BATCH, _FEATURES), jnp.float32),
            pltpu.VMEM((_NBUF, _CHUNK, _FEATURES), jnp.float32),
            pltpu.VMEM((_BATCH, 1), jnp.float32),
            pltpu.VMEM((_BATCH, 1), jnp.float32),
            pltpu.VMEM((_BATCH, 1), jnp.float32),
            pltpu.VMEM((_BATCH, 8), jnp.float32),
            pltpu.SemaphoreType.DMA((_NBUF, _NSPLIT)),
        ],
    )(inputs, lab2d, em, em)
    return out[0, 0]


def kernel(inputs, label, epoch, em):
    del epoch
    return _run(inputs, label, em)
